# baseline (speedup 1.0000x reference)
import functools
import jax
import jax.numpy as jnp
from jax.experimental import pallas as pl


def _edge_kernel(a_ref, b_ref, ef_ref, we_ref, g_ref, bb_ref, t_ref):
    h = a_ref[...] + b_ref[...] + ef_ref[...] * we_ref[...]
    m = jnp.mean(h, axis=-1, keepdims=True)
    v = jnp.mean((h - m) ** 2, axis=-1, keepdims=True)
    t = (h - m) / jnp.sqrt(v + 1e-5) * g_ref[...] + bb_ref[...]
    t_ref[...] = jnp.maximum(t, 0.0)


def _edge_stage(a_g, b_g, ef, we, g1, b1):
    E, D = a_g.shape
    BE = 2000
    grid = (E // BE,)
    return pl.pallas_call(
        _edge_kernel,
        grid=grid,
        in_specs=[
            pl.BlockSpec((BE, D), lambda i: (i, 0)),
            pl.BlockSpec((BE, D), lambda i: (i, 0)),
            pl.BlockSpec((BE, 1), lambda i: (i, 0)),
            pl.BlockSpec((1, D), lambda i: (0, 0)),
            pl.BlockSpec((1, D), lambda i: (0, 0)),
            pl.BlockSpec((1, D), lambda i: (0, 0)),
        ],
        out_specs=pl.BlockSpec((BE, D), lambda i: (i, 0)),
        out_shape=jax.ShapeDtypeStruct((E, D), jnp.float32),
    )(a_g, b_g, ef, we.reshape(1, D), g1.reshape(1, D), b1.reshape(1, D))


def _node_kernel(T_ref, deg_ref, right_ref, Wf_ref, bf_ref, g2_ref, b2_ref,
                 Wo1a_ref, Wo1b_ref, bo1_ref, Wo2_ref, bo2_ref, out_ref):
    aggr = jnp.dot(T_ref[...], Wf_ref[...], preferred_element_type=jnp.float32)
    aggr = aggr + deg_ref[...] * bf_ref[...]
    m = jnp.mean(aggr, axis=-1, keepdims=True)
    v = jnp.mean((aggr - m) ** 2, axis=-1, keepdims=True)
    post = (aggr - m) / jnp.sqrt(v + 1e-5) * g2_ref[...] + b2_ref[...]
    hid = (jnp.dot(post, Wo1a_ref[...], preferred_element_type=jnp.float32)
           + jnp.dot(right_ref[...], Wo1b_ref[...], preferred_element_type=jnp.float32)
           + bo1_ref[...])
    hid = jnp.maximum(hid, 0.0)
    out = jnp.dot(hid, Wo2_ref[...], preferred_element_type=jnp.float32) + bo2_ref[...]
    out_ref[...] = right_ref[...] + out


def _node_stage(T, deg, right, Wf, bf, g2, b2, Wo1, bo1, Wo2, bo2):
    N, D = right.shape
    BN = 1000
    grid = (N // BN,)
    row = lambda i: (i, 0)
    full = lambda i: (0, 0)
    return pl.pallas_call(
        _node_kernel,
        grid=grid,
        in_specs=[
            pl.BlockSpec((BN, D), row),
            pl.BlockSpec((BN, 1), row),
            pl.BlockSpec((BN, D), row),
            pl.BlockSpec((D, D), full),
            pl.BlockSpec((1, D), full),
            pl.BlockSpec((1, D), full),
            pl.BlockSpec((1, D), full),
            pl.BlockSpec((D, D), full),
            pl.BlockSpec((D, D), full),
            pl.BlockSpec((1, D), full),
            pl.BlockSpec((D, D), full),
            pl.BlockSpec((1, D), full),
        ],
        out_specs=pl.BlockSpec((BN, D), row),
        out_shape=jax.ShapeDtypeStruct((N, D), jnp.float32),
    )(T, deg, right, Wf, bf.reshape(1, D), g2.reshape(1, D), b2.reshape(1, D),
      Wo1[:D], Wo1[D:], bo1.reshape(1, D), Wo2, bo2.reshape(1, D))


def _bconv(left, right, src, dst, ef, Wl, bl, We, Wr, g1, b1, Wf, bf, g2, b2,
           Wo1, bo1, Wo2, bo2):
    N, D = right.shape
    A = right @ Wl + bl
    B = left @ Wr
    t = _edge_stage(A[dst], B[src], ef, We[0], g1, b1)
    T = jnp.zeros((N, D), jnp.float32).at[dst].add(t)
    deg = jnp.zeros((N,), jnp.float32).at[dst].add(1.0)
    return _node_stage(T, deg[:, None], right, Wf, bf, g2, b2, Wo1, bo1, Wo2, bo2)


def kernel(constraint_features, edge_indices, edge_features, variable_features,
           W_left, b_left, W_edge, W_right, ln1_g, ln1_b, W_final, b_final,
           ln2_g, ln2_b, W_o1, b_o1, W_o2, b_o2):
    cf, vf = constraint_features, variable_features
    ei0, ei1 = edge_indices[0], edge_indices[1]
    ef = edge_features
    for i in range(2):
        j = 2 * i
        cf = _bconv(vf, cf, ei1, ei0, ef, W_left[j], b_left[j], W_edge[j],
                    W_right[j], ln1_g[j], ln1_b[j], W_final[j], b_final[j],
                    ln2_g[j], ln2_b[j], W_o1[j], b_o1[j], W_o2[j], b_o2[j])
        j = 2 * i + 1
        vf = _bconv(cf, vf, ei0, ei1, ef, W_left[j], b_left[j], W_edge[j],
                    W_right[j], ln1_g[j], ln1_b[j], W_final[j], b_final[j],
                    ln2_g[j], ln2_b[j], W_o1[j], b_o1[j], W_o2[j], b_o2[j])
    return (cf, vf)
